# baseline (speedup 1.0000x reference)
import functools

import jax
import jax.numpy as jnp
from jax import lax
from jax.experimental import pallas as pl
from jax.experimental.pallas import tpu as pltpu
from jax.experimental.pallas import tpu_sc as plsc

NC = 2
NS = 16
NW = NC * NS
CHUNK = 40
NBUF = 5
ZROWS = 25
BLK = 1000


def _sc_mesh():
    return plsc.VectorSubcoreMesh(
        core_axis_name="c", subcore_axis_name="s", num_cores=NC, num_subcores=NS
    )


def _make_sc_agg(n_nodes, gcols, d_edge, n_edges, do_edges):
    epw = n_edges // NW
    n_chunks = epw // CHUNK
    n_mid = n_chunks // NBUF - 2
    n_tail = n_chunks - NBUF * (1 + n_mid)
    assert n_mid >= 1 and n_tail >= NBUF
    rows_per_sub = n_nodes // NS

    out_type = [jax.ShapeDtypeStruct((NC, n_nodes, gcols), jnp.float32)]
    scratch = [
        pltpu.VMEM((NBUF, CHUNK), jnp.int32),
        pltpu.VMEM((NBUF, CHUNK), jnp.int32),
        pltpu.VMEM((NBUF, CHUNK, gcols), jnp.float32),
        pltpu.VMEM_SHARED((n_nodes, gcols), jnp.float32),
        pltpu.SemaphoreType.DMA((NBUF,)),
        pltpu.SemaphoreType.DMA((NBUF,)),
        pltpu.SemaphoreType.DMA((NBUF,)),
        pltpu.SemaphoreType.DMA((NBUF,)),
    ]
    if do_edges:
        out_type.append(jax.ShapeDtypeStruct((NC, n_nodes, d_edge), jnp.float32))
        scratch += [
            pltpu.VMEM((NBUF, CHUNK, d_edge), jnp.float32),
            pltpu.VMEM_SHARED((n_nodes, d_edge), jnp.float32),
            pltpu.SemaphoreType.DMA((NBUF,)),
            pltpu.SemaphoreType.DMA((NBUF,)),
        ]

    def body(*refs):
        if do_edges:
            (table, src3, dst3, zg, eattr,
             out_g, out_e,
             sbuf, dbuf, rows, acc_g, issem, idsem, gsem, ssem,
             erows, acc_e, lsem, tsem) = refs
        else:
            (table, src3, dst3, zg,
             out_g,
             sbuf, dbuf, rows, acc_g, issem, idsem, gsem, ssem) = refs
        c = lax.axis_index("c")
        s = lax.axis_index("s")
        w = s * NC + c

        r0 = s * rows_per_sub
        pltpu.sync_copy(zg.at[:, pl.ds(0, gcols)],
                        acc_g.at[pl.ds(r0, rows_per_sub), :])
        if do_edges:
            pltpu.sync_copy(zg.at[:, pl.ds(0, d_edge)],
                            acc_e.at[pl.ds(r0, rows_per_sub), :])
        plsc.subcore_barrier()

        base0 = w * epw

        def issue_is(p, b):
            pltpu.async_copy(src3.at[w, p], sbuf.at[b], issem.at[b])

        def wait_is(p, b):
            pltpu.make_async_copy(src3.at[w, p], sbuf.at[b], issem.at[b]).wait()

        def issue_id(p, b):
            pltpu.async_copy(dst3.at[w, p], dbuf.at[b], idsem.at[b])

        def wait_id(p, b):
            pltpu.make_async_copy(dst3.at[w, p], dbuf.at[b], idsem.at[b]).wait()

        def issue_gather(p, b):
            pltpu.async_copy(table.at[sbuf.at[b]], rows.at[b], gsem.at[b])
            if do_edges:
                pltpu.async_copy(eattr.at[pl.ds(base0 + p * CHUNK, CHUNK), :],
                                 erows.at[b], lsem.at[b])

        def wait_gather(p, b):
            pltpu.make_async_copy(table.at[sbuf.at[b]], rows.at[b],
                                  gsem.at[b]).wait()
            if do_edges:
                pltpu.make_async_copy(eattr.at[pl.ds(base0 + p * CHUNK, CHUNK), :],
                                      erows.at[b], lsem.at[b]).wait()

        def issue_scatter(j, b):
            pltpu.async_copy(rows.at[b], acc_g.at[dbuf.at[b]], ssem.at[b],
                             add=True)
            if do_edges:
                pltpu.async_copy(erows.at[b], acc_e.at[dbuf.at[b]], tsem.at[b],
                                 add=True)

        def wait_scatter(j, b):
            pltpu.make_async_copy(rows.at[b], acc_g.at[dbuf.at[b]],
                                  ssem.at[b]).wait()
            if do_edges:
                pltpu.make_async_copy(erows.at[b], acc_e.at[dbuf.at[b]],
                                      tsem.at[b]).wait()

        def slot(j, b, with_wait_s, with_next, with_is):
            b2 = (b + 2) % NBUF
            b4 = (b + 4) % NBUF
            wait_gather(j, b)
            wait_id(j, b)
            issue_scatter(j, b)
            if with_wait_s:
                wait_scatter(j - (NBUF - 2), b2)
            if with_next:
                wait_is(j + 2, b2)
                issue_gather(j + 2, b2)
                issue_id(j + 2, b2)
            if with_is:
                issue_is(j + 4, b4)

        for k in range(4):
            issue_is(k, k)
        for k in range(2):
            issue_id(k, k)
            wait_is(k, k)
            issue_gather(k, k)

        for b in range(NBUF):
            slot(b, b, b >= NBUF - 2, True, True)

        def outer(jj, carry):
            for b in range(NBUF):
                slot(jj * NBUF + b, b, True, True, True)
            return carry

        lax.fori_loop(1, 1 + n_mid, outer, 0)

        for k in range(n_tail):
            j = NBUF * (1 + n_mid) + k
            slot(j, j % NBUF, (j + 2) < n_chunks, (j + 2) < n_chunks,
                 (j + 4) < n_chunks)
        for j in range(n_chunks - NBUF, n_chunks):
            wait_scatter(j, j % NBUF)
        plsc.subcore_barrier()

        pltpu.sync_copy(acc_g.at[pl.ds(r0, rows_per_sub), :],
                        out_g.at[c, pl.ds(r0, rows_per_sub), :])
        if do_edges:
            pltpu.sync_copy(acc_e.at[pl.ds(r0, rows_per_sub), :],
                            out_e.at[c, pl.ds(r0, rows_per_sub), :])

    return pl.kernel(
        body, out_type=out_type, mesh=_sc_mesh(), scratch_types=scratch,
        compiler_params=pltpu.CompilerParams(use_tc_tiling_on_sc=False))


def _fp_body(attrs_ref, w_out_ref, b_out_ref, fp_ref):
    logits = (jnp.dot(attrs_ref[...].astype(jnp.bfloat16),
                      w_out_ref[...].astype(jnp.bfloat16),
                      preferred_element_type=jnp.float32) + b_out_ref[...])
    m = jnp.max(logits, axis=-1, keepdims=True)
    p = jnp.exp(logits - m)
    srow = jnp.sum(p, axis=-1, keepdims=True)
    contrib = jnp.sum(p / srow, axis=0, keepdims=True)

    @pl.when(pl.program_id(0) == 0)
    def _():
        fp_ref[...] = jnp.zeros_like(fp_ref)

    fp_ref[...] += contrib


def _make_fp(n_nodes, cols, inner, out_dim):
    grid = n_nodes // BLK
    return pl.pallas_call(
        _fp_body,
        grid=(grid,),
        in_specs=[
            pl.BlockSpec((BLK, cols), lambda i: (i, 0)),
            pl.BlockSpec((cols, out_dim), lambda i: (0, 0)),
            pl.BlockSpec((1, out_dim), lambda i: (0, 0)),
        ],
        out_specs=pl.BlockSpec((1, out_dim), lambda i: (0, 0)),
        out_shape=jax.ShapeDtypeStruct((1, out_dim), jnp.float32),
    )


def _inner_body(d_node, first, attrs_ref, aggm_ref, agge_ref,
                w_in_ref, b_in_ref, new_ref):
    v = attrs_ref[...] + aggm_ref[0] + aggm_ref[1]
    ve_agg = agge_ref[0] + agge_ref[1]
    if first:
        ve = ve_agg
        vh = v
    else:
        ve = v[:, d_node:] + ve_agg
        vh = v[:, :d_node]
    v = jnp.concatenate([vh, ve], axis=1)
    new_ref[...] = (jnp.dot(v, w_in_ref[...], preferred_element_type=jnp.float32)
                    + b_in_ref[...])


def _make_inner(n_nodes, d_node, d_edge, inner, first):
    grid = n_nodes // BLK
    acols = d_node if first else inner
    return pl.pallas_call(
        functools.partial(_inner_body, d_node, first),
        grid=(grid,),
        in_specs=[
            pl.BlockSpec((BLK, acols), lambda i: (i, 0)),
            pl.BlockSpec((NC, BLK, acols), lambda i: (0, i, 0)),
            pl.BlockSpec((NC, BLK, d_edge), lambda i: (0, i, 0)),
            pl.BlockSpec((inner, inner), lambda i: (0, 0)),
            pl.BlockSpec((1, inner), lambda i: (0, 0)),
        ],
        out_specs=pl.BlockSpec((BLK, inner), lambda i: (i, 0)),
        out_shape=jax.ShapeDtypeStruct((n_nodes, inner), jnp.float32),
    )


def kernel(node_attr, edge_index, edge_attr, W_inner, b_inner, W_output, b_output):
    n_nodes, d_node = node_attr.shape
    n_edges, d_edge = edge_attr.shape
    inner = d_node + d_edge
    depth = W_inner.shape[0] - 1
    out_dim = W_output.shape[2]

    epw = n_edges // NW
    n_chunks = epw // CHUNK
    src3 = edge_index[0].reshape(NW, n_chunks, CHUNK)
    dst3 = edge_index[1].reshape(NW, n_chunks, CHUNK)
    zeros_tile = jnp.zeros((n_nodes // NS, inner), jnp.float32)

    sc_agg_first = _make_sc_agg(n_nodes, d_node, d_edge, n_edges, True)
    sc_agg = _make_sc_agg(n_nodes, inner, d_edge, n_edges, False)
    fp0_call = _make_fp(n_nodes, d_node, inner, out_dim)
    fp_call = _make_fp(n_nodes, inner, inner, out_dim)
    inner_first = _make_inner(n_nodes, d_node, d_edge, inner, True)
    inner_call = _make_inner(n_nodes, d_node, d_edge, inner, False)

    agg_g, agg_e = sc_agg_first(node_attr, src3, dst3, zeros_tile, edge_attr)
    fp = fp0_call(node_attr, W_output[0], b_output[0])
    attrs = inner_first(node_attr, agg_g, agg_e, W_inner[1], b_inner[1])
    for d in range(2, depth + 1):
        (agg_m,) = sc_agg(attrs, src3, dst3, zeros_tile)
        fp = fp + fp_call(attrs, W_output[d - 1], b_output[d - 1])
        attrs = inner_call(attrs, agg_m, agg_e, W_inner[d], b_inner[d])
    fp = fp + fp_call(attrs, W_output[depth], b_output[depth])
    return fp.reshape(out_dim)


# speedup vs baseline: 1.0012x; 1.0012x over previous
import functools

import jax
import jax.numpy as jnp
from jax import lax
from jax.experimental import pallas as pl
from jax.experimental.pallas import tpu as pltpu
from jax.experimental.pallas import tpu_sc as plsc

NC = 2
NS = 16
NW = NC * NS
CHUNK = 40
NBUF = 5
ZROWS = 25
BLK = 1000


def _sc_mesh():
    return plsc.VectorSubcoreMesh(
        core_axis_name="c", subcore_axis_name="s", num_cores=NC, num_subcores=NS
    )


def _make_sc_agg(n_nodes, gcols, d_edge, n_edges, do_edges):
    epw = n_edges // NW
    n_chunks = epw // CHUNK
    n_mid = n_chunks // NBUF - 2
    n_tail = n_chunks - NBUF * (1 + n_mid)
    assert n_mid >= 1 and n_tail >= NBUF
    rows_per_sub = n_nodes // NS

    out_type = [jax.ShapeDtypeStruct((NC, n_nodes, gcols), jnp.float32)]
    scratch = [
        pltpu.VMEM((NBUF, CHUNK), jnp.int32),
        pltpu.VMEM((NBUF, CHUNK), jnp.int32),
        pltpu.VMEM((NBUF, CHUNK, gcols), jnp.float32),
        pltpu.VMEM_SHARED((n_nodes, gcols), jnp.float32),
        pltpu.SemaphoreType.DMA((NBUF,)),
        pltpu.SemaphoreType.DMA((NBUF,)),
        pltpu.SemaphoreType.DMA((NBUF,)),
        pltpu.SemaphoreType.DMA((NBUF,)),
    ]
    if do_edges:
        out_type.append(jax.ShapeDtypeStruct((NC, n_nodes, d_edge), jnp.float32))
        scratch += [
            pltpu.VMEM((NBUF, CHUNK, d_edge), jnp.float32),
            pltpu.VMEM_SHARED((n_nodes, d_edge), jnp.float32),
            pltpu.SemaphoreType.DMA((NBUF,)),
            pltpu.SemaphoreType.DMA((NBUF,)),
        ]

    def body(*refs):
        if do_edges:
            (table, src3, dst3, zg, eattr,
             out_g, out_e,
             sbuf, dbuf, rows, acc_g, issem, idsem, gsem, ssem,
             erows, acc_e, lsem, tsem) = refs
        else:
            (table, src3, dst3, zg,
             out_g,
             sbuf, dbuf, rows, acc_g, issem, idsem, gsem, ssem) = refs
        c = lax.axis_index("c")
        s = lax.axis_index("s")
        w = s * NC + c

        r0 = s * rows_per_sub
        pltpu.sync_copy(zg.at[:, pl.ds(0, gcols)],
                        acc_g.at[pl.ds(r0, rows_per_sub), :])
        if do_edges:
            pltpu.sync_copy(zg.at[:, pl.ds(0, d_edge)],
                            acc_e.at[pl.ds(r0, rows_per_sub), :])
        plsc.subcore_barrier()

        base0 = w * epw

        def issue_is(p, b):
            pltpu.async_copy(src3.at[w, p], sbuf.at[b], issem.at[b])

        def wait_is(p, b):
            pltpu.make_async_copy(src3.at[w, p], sbuf.at[b], issem.at[b]).wait()

        def issue_id(p, b):
            pltpu.async_copy(dst3.at[w, p], dbuf.at[b], idsem.at[b])

        def wait_id(p, b):
            pltpu.make_async_copy(dst3.at[w, p], dbuf.at[b], idsem.at[b]).wait()

        def issue_gather(p, b):
            pltpu.async_copy(table.at[sbuf.at[b]], rows.at[b], gsem.at[b])
            if do_edges:
                pltpu.async_copy(eattr.at[pl.ds(base0 + p * CHUNK, CHUNK), :],
                                 erows.at[b], lsem.at[b])

        def wait_gather(p, b):
            pltpu.make_async_copy(table.at[sbuf.at[b]], rows.at[b],
                                  gsem.at[b]).wait()
            if do_edges:
                pltpu.make_async_copy(eattr.at[pl.ds(base0 + p * CHUNK, CHUNK), :],
                                      erows.at[b], lsem.at[b]).wait()

        def issue_scatter(j, b):
            pltpu.async_copy(rows.at[b], acc_g.at[dbuf.at[b]], ssem.at[b],
                             add=True)
            if do_edges:
                pltpu.async_copy(erows.at[b], acc_e.at[dbuf.at[b]], tsem.at[b],
                                 add=True)

        def wait_scatter(j, b):
            pltpu.make_async_copy(rows.at[b], acc_g.at[dbuf.at[b]],
                                  ssem.at[b]).wait()
            if do_edges:
                pltpu.make_async_copy(erows.at[b], acc_e.at[dbuf.at[b]],
                                      tsem.at[b]).wait()

        def slot(j, b, with_wait_s, with_next, with_is):
            b2 = (b + 2) % NBUF
            b4 = (b + 4) % NBUF
            wait_gather(j, b)
            wait_id(j, b)
            issue_scatter(j, b)
            if with_wait_s:
                wait_scatter(j - (NBUF - 2), b2)
            if with_next:
                wait_is(j + 2, b2)
                issue_gather(j + 2, b2)
                issue_id(j + 2, b2)
            if with_is:
                issue_is(j + 4, b4)

        for k in range(4):
            issue_is(k, k)
        for k in range(2):
            issue_id(k, k)
            wait_is(k, k)
            issue_gather(k, k)

        for b in range(NBUF):
            slot(b, b, b >= NBUF - 2, True, True)

        def outer(jj, carry):
            for b in range(NBUF):
                slot(jj * NBUF + b, b, True, True, True)
            return carry

        lax.fori_loop(1, 1 + n_mid, outer, 0)

        for k in range(n_tail):
            j = NBUF * (1 + n_mid) + k
            slot(j, j % NBUF, (j + 2) < n_chunks, (j + 2) < n_chunks,
                 (j + 4) < n_chunks)
        for j in range(n_chunks - NBUF, n_chunks):
            wait_scatter(j, j % NBUF)
        plsc.subcore_barrier()

        pltpu.sync_copy(acc_g.at[pl.ds(r0, rows_per_sub), :],
                        out_g.at[c, pl.ds(r0, rows_per_sub), :])
        if do_edges:
            pltpu.sync_copy(acc_e.at[pl.ds(r0, rows_per_sub), :],
                            out_e.at[c, pl.ds(r0, rows_per_sub), :])

    return pl.kernel(
        body, out_type=out_type, mesh=_sc_mesh(), scratch_types=scratch,
        compiler_params=pltpu.CompilerParams(use_tc_tiling_on_sc=False))


def _fp_body(attrs_ref, w_out_ref, b_out_ref, fp_ref):
    logits = (jnp.dot(attrs_ref[...], w_out_ref[...],
                      preferred_element_type=jnp.float32) + b_out_ref[...])
    m = jnp.max(logits, axis=-1, keepdims=True)
    p = jnp.exp(logits - m)
    srow = jnp.sum(p, axis=-1, keepdims=True)
    contrib = jnp.sum(p / srow, axis=0, keepdims=True)

    @pl.when(pl.program_id(0) == 0)
    def _():
        fp_ref[...] = jnp.zeros_like(fp_ref)

    fp_ref[...] += contrib


def _make_fp(n_nodes, cols, inner, out_dim):
    grid = n_nodes // BLK
    return pl.pallas_call(
        _fp_body,
        grid=(grid,),
        in_specs=[
            pl.BlockSpec((BLK, cols), lambda i: (i, 0)),
            pl.BlockSpec((cols, out_dim), lambda i: (0, 0)),
            pl.BlockSpec((1, out_dim), lambda i: (0, 0)),
        ],
        out_specs=pl.BlockSpec((1, out_dim), lambda i: (0, 0)),
        out_shape=jax.ShapeDtypeStruct((1, out_dim), jnp.float32),
    )


def _inner_body(d_node, first, attrs_ref, aggm_ref, agge_ref,
                w_in_ref, b_in_ref, new_ref):
    v = attrs_ref[...] + aggm_ref[0] + aggm_ref[1]
    ve_agg = agge_ref[0] + agge_ref[1]
    if first:
        ve = ve_agg
        vh = v
    else:
        ve = v[:, d_node:] + ve_agg
        vh = v[:, :d_node]
    v = jnp.concatenate([vh, ve], axis=1)
    new_ref[...] = (jnp.dot(v, w_in_ref[...], preferred_element_type=jnp.float32)
                    + b_in_ref[...])


def _make_inner(n_nodes, d_node, d_edge, inner, first):
    grid = n_nodes // BLK
    acols = d_node if first else inner
    return pl.pallas_call(
        functools.partial(_inner_body, d_node, first),
        grid=(grid,),
        in_specs=[
            pl.BlockSpec((BLK, acols), lambda i: (i, 0)),
            pl.BlockSpec((NC, BLK, acols), lambda i: (0, i, 0)),
            pl.BlockSpec((NC, BLK, d_edge), lambda i: (0, i, 0)),
            pl.BlockSpec((inner, inner), lambda i: (0, 0)),
            pl.BlockSpec((1, inner), lambda i: (0, 0)),
        ],
        out_specs=pl.BlockSpec((BLK, inner), lambda i: (i, 0)),
        out_shape=jax.ShapeDtypeStruct((n_nodes, inner), jnp.float32),
    )


def kernel(node_attr, edge_index, edge_attr, W_inner, b_inner, W_output, b_output):
    n_nodes, d_node = node_attr.shape
    n_edges, d_edge = edge_attr.shape
    inner = d_node + d_edge
    depth = W_inner.shape[0] - 1
    out_dim = W_output.shape[2]

    epw = n_edges // NW
    n_chunks = epw // CHUNK
    src3 = edge_index[0].reshape(NW, n_chunks, CHUNK)
    dst3 = edge_index[1].reshape(NW, n_chunks, CHUNK)
    zeros_tile = jnp.zeros((n_nodes // NS, inner), jnp.float32)

    sc_agg_first = _make_sc_agg(n_nodes, d_node, d_edge, n_edges, True)
    sc_agg = _make_sc_agg(n_nodes, inner, d_edge, n_edges, False)
    fp0_call = _make_fp(n_nodes, d_node, inner, out_dim)
    fp_call = _make_fp(n_nodes, inner, inner, out_dim)
    inner_first = _make_inner(n_nodes, d_node, d_edge, inner, True)
    inner_call = _make_inner(n_nodes, d_node, d_edge, inner, False)

    agg_g, agg_e = sc_agg_first(node_attr, src3, dst3, zeros_tile, edge_attr)
    fp = fp0_call(node_attr, W_output[0], b_output[0])
    attrs = inner_first(node_attr, agg_g, agg_e, W_inner[1], b_inner[1])
    for d in range(2, depth + 1):
        (agg_m,) = sc_agg(attrs, src3, dst3, zeros_tile)
        fp = fp + fp_call(attrs, W_output[d - 1], b_output[d - 1])
        attrs = inner_call(attrs, agg_m, agg_e, W_inner[d], b_inner[d])
    fp = fp + fp_call(attrs, W_output[depth], b_output[depth])
    return fp.reshape(out_dim)
